# baseline (speedup 1.0000x reference)
import jax
import jax.numpy as jnp
from jax.experimental import pallas as pl
from jax.experimental.pallas import tpu as pltpu

_D_MODEL = 768
_DG = 3072
_MEM = 50000
_KS = 61
_TOPK = 5
_BLK = 1000
_NBLK = _MEM // _BLK


def _hm_kernel(q_ref, w_ref, b_ref, keys_ref, imp_ref, vals_ref,
               retr_ref, sims_ref,
               sparse_scr, simsall_scr, sem):
    i = pl.program_id(0)

    @pl.when(i == 0)
    def _prologue():
        q = q_ref[...]
        w = w_ref[...]
        expanded = jnp.maximum(
            jnp.dot(q, w, preferred_element_type=jnp.float32) + b_ref[...],
            0.0)
        bits = jax.lax.bitcast_convert_type(expanded, jnp.int32)

        def body(_, carry):
            lo, hi = carry
            mid = lo + (hi - lo) // 2
            cnt = jnp.sum((bits >= mid).astype(jnp.int32))
            ge = cnt >= _KS
            return jnp.where(ge, mid, lo), jnp.where(ge, hi, mid)

        lo, _ = jax.lax.fori_loop(
            0, 31, body, (jnp.int32(0), jnp.int32(0x7F800000)))
        sparse_scr[...] = jnp.where(bits >= lo, expanded, 0.0)

    sparse = sparse_scr[...]
    keys = keys_ref[...]
    dots = jax.lax.dot_general(
        sparse, keys, (((1,), (1,)), ((), ())),
        preferred_element_type=jnp.float32)
    keys_bf = keys.astype(jnp.bfloat16)
    ones = jnp.ones((1, _DG), jnp.bfloat16)
    sq = jax.lax.dot_general(
        ones, keys_bf * keys_bf, (((1,), (1,)), ((), ())),
        preferred_element_type=jnp.float32)
    qn = jnp.maximum(jnp.sqrt(jnp.sum(sparse * sparse)), 1e-8)
    kn = jnp.maximum(jnp.sqrt(sq), 1e-8)
    imp = imp_ref[pl.ds(i, 1), :]
    simsall_scr[pl.ds(i, 1), :] = dots * imp / (kn * qn)

    @pl.when(i == _NBLK - 1)
    def _epilogue():
        s = simsall_scr[...]
        row = jax.lax.broadcasted_iota(jnp.int32, (_NBLK, _BLK), 0)
        col = jax.lax.broadcasted_iota(jnp.int32, (_NBLK, _BLK), 1)
        flat = row * _BLK + col
        lane = jax.lax.broadcasted_iota(jnp.int32, (1, 128), 1)
        out_vec = jnp.zeros((1, 128), jnp.float32)
        for j in range(_TOPK):
            m = jnp.max(s)
            cand = jnp.where(s == m, flat, jnp.int32(2**30))
            idx = jnp.min(cand)
            cp = pltpu.make_async_copy(
                vals_ref.at[pl.ds(idx, 1), :],
                retr_ref.at[pl.ds(j, 1), :], sem)
            cp.start()
            cp.wait()
            out_vec = out_vec + jnp.where(lane == j, m, 0.0)
            s = jnp.where(flat == idx, -jnp.inf, s)
        sims_ref[...] = out_vec


def kernel(query, W_dg, b_dg, ca3_keys, ca3_values, importance, k):
    q2 = query.reshape(1, _D_MODEL)
    b2 = b_dg.reshape(1, _DG)
    imp2 = importance.reshape(_NBLK, _BLK)
    retr, sims = pl.pallas_call(
        _hm_kernel,
        grid=(_NBLK,),
        in_specs=[
            pl.BlockSpec((1, _D_MODEL), lambda i: (0, 0)),
            pl.BlockSpec((_D_MODEL, _DG), lambda i: (0, 0)),
            pl.BlockSpec((1, _DG), lambda i: (0, 0)),
            pl.BlockSpec((_BLK, _DG), lambda i: (i, 0)),
            pl.BlockSpec((_NBLK, _BLK), lambda i: (0, 0)),
            pl.BlockSpec(memory_space=pltpu.MemorySpace.HBM),
        ],
        out_specs=[
            pl.BlockSpec((_TOPK, _D_MODEL), lambda i: (0, 0)),
            pl.BlockSpec((1, 128), lambda i: (0, 0)),
        ],
        out_shape=[
            jax.ShapeDtypeStruct((_TOPK, _D_MODEL), jnp.float32),
            jax.ShapeDtypeStruct((1, 128), jnp.float32),
        ],
        scratch_shapes=[
            pltpu.VMEM((1, _DG), jnp.float32),
            pltpu.VMEM((_NBLK, _BLK), jnp.float32),
            pltpu.SemaphoreType.DMA,
        ],
        compiler_params=pltpu.CompilerParams(
            dimension_semantics=("arbitrary",)),
    )(q2, W_dg, b2, ca3_keys, imp2, ca3_values)
    top_sim = sims[0, :_TOPK] + (jnp.asarray(k) * 0).astype(jnp.float32)
    return retr, top_sim


# speedup vs baseline: 1.0335x; 1.0335x over previous
import jax
import jax.numpy as jnp
from jax.experimental import pallas as pl
from jax.experimental.pallas import tpu as pltpu

_D_MODEL = 768
_DG = 3072
_MEM = 50000
_KS = 61
_TOPK = 5
_BLK = 2000
_NBLK = _MEM // _BLK


def _dg_kernel(q_ref, w_ref, b_ref, sparse_ref):
    q = q_ref[...]
    w = w_ref[...]
    expanded = jnp.maximum(
        jnp.dot(q, w, preferred_element_type=jnp.float32) + b_ref[...],
        0.0)
    bits = jax.lax.bitcast_convert_type(expanded, jnp.int32)

    def body(_, carry):
        lo, hi = carry
        mid = lo + (hi - lo) // 2
        cnt = jnp.sum((bits >= mid).astype(jnp.int32))
        ge = cnt >= _KS
        return jnp.where(ge, mid, lo), jnp.where(ge, hi, mid)

    lo, _ = jax.lax.fori_loop(
        0, 31, body, (jnp.int32(0), jnp.int32(0x7F800000)))
    sparse = jnp.where(bits >= lo, expanded, 0.0)
    qn = jnp.maximum(jnp.sqrt(jnp.sum(sparse * sparse)), 1e-8)
    sparse_ref[...] = sparse / qn


def _scan_kernel(sparse_ref, keys_ref, imp_ref, sims_ref):
    i = pl.program_id(0)
    sparse = sparse_ref[...]
    keys = keys_ref[...]
    dots = jax.lax.dot_general(
        sparse, keys, (((1,), (1,)), ((), ())),
        preferred_element_type=jnp.float32)
    keys_bf = keys.astype(jnp.bfloat16)
    ones = jnp.ones((1, _DG), jnp.bfloat16)
    sq = jax.lax.dot_general(
        ones, keys_bf * keys_bf, (((1,), (1,)), ((), ())),
        preferred_element_type=jnp.float32)
    kn = jnp.maximum(jnp.sqrt(sq), 1e-8)
    imp = imp_ref[pl.ds(i, 1), :]
    sims_ref[0, :, :] = dots * imp / kn


def _top_kernel(sims_ref, vals_ref, retr_ref, top_ref, sem):
    s = sims_ref[...]
    row = jax.lax.broadcasted_iota(jnp.int32, (_NBLK, _BLK), 0)
    col = jax.lax.broadcasted_iota(jnp.int32, (_NBLK, _BLK), 1)
    flat = row * _BLK + col
    lane = jax.lax.broadcasted_iota(jnp.int32, (1, 128), 1)
    out_vec = jnp.zeros((1, 128), jnp.float32)
    for j in range(_TOPK):
        m = jnp.max(s)
        cand = jnp.where(s == m, flat, jnp.int32(2**30))
        idx = jnp.min(cand)
        cp = pltpu.make_async_copy(
            vals_ref.at[pl.ds(idx, 1), :],
            retr_ref.at[pl.ds(j, 1), :], sem)
        cp.start()
        cp.wait()
        out_vec = out_vec + jnp.where(lane == j, m, 0.0)
        s = jnp.where(flat == idx, -jnp.inf, s)
    top_ref[...] = out_vec


def kernel(query, W_dg, b_dg, ca3_keys, ca3_values, importance, k):
    q2 = query.reshape(1, _D_MODEL)
    b2 = b_dg.reshape(1, _DG)
    imp2 = importance.reshape(_NBLK, _BLK)
    sparse = pl.pallas_call(
        _dg_kernel,
        out_shape=jax.ShapeDtypeStruct((1, _DG), jnp.float32),
    )(q2, W_dg, b2)
    sims = pl.pallas_call(
        _scan_kernel,
        grid=(_NBLK,),
        in_specs=[
            pl.BlockSpec((1, _DG), lambda i: (0, 0)),
            pl.BlockSpec((_BLK, _DG), lambda i: (i, 0)),
            pl.BlockSpec((_NBLK, _BLK), lambda i: (0, 0)),
        ],
        out_specs=pl.BlockSpec((1, 1, _BLK), lambda i: (i, 0, 0)),
        out_shape=jax.ShapeDtypeStruct((_NBLK, 1, _BLK), jnp.float32),
        compiler_params=pltpu.CompilerParams(
            dimension_semantics=("arbitrary",)),
    )(sparse, ca3_keys, imp2)
    sims = sims.reshape(_NBLK, _BLK)
    retr, top = pl.pallas_call(
        _top_kernel,
        in_specs=[
            pl.BlockSpec((_NBLK, _BLK), lambda: (0, 0)),
            pl.BlockSpec(memory_space=pltpu.MemorySpace.HBM),
        ],
        out_specs=[
            pl.BlockSpec((_TOPK, _D_MODEL), lambda: (0, 0)),
            pl.BlockSpec((1, 128), lambda: (0, 0)),
        ],
        out_shape=[
            jax.ShapeDtypeStruct((_TOPK, _D_MODEL), jnp.float32),
            jax.ShapeDtypeStruct((1, 128), jnp.float32),
        ],
        scratch_shapes=[pltpu.SemaphoreType.DMA],
    )(sims, ca3_values)
    top_sim = top[0, :_TOPK] + (jnp.asarray(k) * 0).astype(jnp.float32)
    return retr, top_sim
